# Initial kernel scaffold; baseline (speedup 1.0000x reference)
import jax
import jax.numpy as jnp
from jax.experimental import pallas as pl


def kernel(h, batch, W_f, b_f, W_g, b_g):
    raise NotImplementedError("write your pallas kernel here")


# speedup vs baseline: 1.8716x; 1.8716x over previous
import functools

import jax
import jax.numpy as jnp
from jax import lax
from jax.experimental import pallas as pl
from jax.experimental.pallas import tpu as pltpu
from jax.experimental.pallas import tpu_sc as plsc

NUM_GRAPHS = 1024


_BM = 512


def _dense_body(h_ref, wf_ref, bf_ref, wg_ref, bg_ref, out_ref):
    hb = h_ref[...]
    t = jnp.dot(hb, wf_ref[...], preferred_element_type=jnp.float32)
    t = t + bf_ref[...]
    s = jnp.dot(hb, wg_ref[...], preferred_element_type=jnp.float32)
    s = s + bg_ref[...]
    g = 1.0 / (1.0 + jnp.exp(-s))
    out_ref[...] = t * g


def _dense_stage(h, W_f, b_f, W_g, b_g):
    n, ndim = h.shape
    gdim = W_f.shape[1]
    grid = (pl.cdiv(n, _BM),)
    return pl.pallas_call(
        _dense_body,
        grid=grid,
        in_specs=[
            pl.BlockSpec((_BM, ndim), lambda i: (i, 0)),
            pl.BlockSpec((ndim, gdim), lambda i: (0, 0)),
            pl.BlockSpec((1, gdim), lambda i: (0, 0)),
            pl.BlockSpec((ndim, 1), lambda i: (0, 0)),
            pl.BlockSpec((1, 1), lambda i: (0, 0)),
        ],
        out_specs=pl.BlockSpec((_BM, gdim), lambda i: (i, 0)),
        out_shape=jax.ShapeDtypeStruct((n, gdim), jnp.float32),
    )(h, W_f, b_f.reshape(1, gdim), W_g, b_g.reshape(1, 1))


_C = 80


def _make_scatter(n, gdim):
    nchunk = n // _C
    rows_per_sub = NUM_GRAPHS // 16
    mesh = plsc.VectorSubcoreMesh(core_axis_name="c", subcore_axis_name="s")

    @functools.partial(
        pl.kernel,
        out_type=jax.ShapeDtypeStruct((2, NUM_GRAPHS, gdim), jnp.float32),
        mesh=mesh,
        scratch_types=[
            pltpu.VMEM((_C,), jnp.int32),
            pltpu.VMEM((_C, gdim), jnp.float32),
            pltpu.VMEM_SHARED((NUM_GRAPHS, gdim), jnp.float32),
        ],
    )
    def _scatter(hg_hbm, idx_hbm, zeros_hbm, out_hbm, idx_v, rows_v, acc):
        cid = lax.axis_index("c")
        sid = lax.axis_index("s")
        wid = sid * 2 + cid

        pltpu.sync_copy(
            zeros_hbm.at[pl.ds(sid * rows_per_sub, rows_per_sub)],
            acc.at[pl.ds(sid * rows_per_sub, rows_per_sub)],
        )
        plsc.subcore_barrier()

        nt = (nchunk - wid + 31) // 32

        def body(t, carry):
            chunk = wid + t * 32
            row0 = chunk * _C
            pltpu.sync_copy(idx_hbm.at[pl.ds(row0, _C)], idx_v)
            pltpu.sync_copy(hg_hbm.at[pl.ds(row0, _C)], rows_v)
            pltpu.sync_copy(rows_v, acc.at[idx_v], add=True)
            return carry

        lax.fori_loop(0, nt, body, 0)
        plsc.subcore_barrier()

        pltpu.sync_copy(
            acc.at[pl.ds(sid * rows_per_sub, rows_per_sub)],
            out_hbm.at[cid, pl.ds(sid * rows_per_sub, rows_per_sub)],
        )

    return _scatter


def _combine_body(p_ref, o_ref):
    o_ref[...] = p_ref[0] + p_ref[1]


def _combine(partials):
    _, ngraphs, gdim = partials.shape
    return pl.pallas_call(
        _combine_body,
        out_shape=jax.ShapeDtypeStruct((ngraphs, gdim), jnp.float32),
    )(partials)


def kernel(h, batch, W_f, b_f, W_g, b_g):
    n, _ = h.shape
    gdim = W_f.shape[1]
    hg = _dense_stage(h, W_f, b_f, W_g, b_g)
    idx = batch.astype(jnp.int32)
    zeros = jnp.zeros((NUM_GRAPHS, gdim), jnp.float32)
    partials = _make_scatter(n, gdim)(hg, idx, zeros)
    return _combine(partials)


# speedup vs baseline: 2.1374x; 1.1420x over previous
import functools

import jax
import jax.numpy as jnp
from jax import lax
from jax.experimental import pallas as pl
from jax.experimental.pallas import tpu as pltpu
from jax.experimental.pallas import tpu_sc as plsc

NUM_GRAPHS = 1024


_BM = 512


def _dense_body(h_ref, wf_ref, bf_ref, wg_ref, bg_ref, out_ref):
    hb = h_ref[...]
    t = jnp.dot(hb, wf_ref[...], preferred_element_type=jnp.float32)
    t = t + bf_ref[...]
    s = jnp.dot(hb, wg_ref[...], preferred_element_type=jnp.float32)
    s = s + bg_ref[...]
    g = 1.0 / (1.0 + jnp.exp(-s))
    out_ref[...] = t * g


def _dense_stage(h, W_f, b_f, W_g, b_g):
    n, ndim = h.shape
    gdim = W_f.shape[1]
    grid = (pl.cdiv(n, _BM),)
    return pl.pallas_call(
        _dense_body,
        grid=grid,
        in_specs=[
            pl.BlockSpec((_BM, ndim), lambda i: (i, 0)),
            pl.BlockSpec((ndim, gdim), lambda i: (0, 0)),
            pl.BlockSpec((1, gdim), lambda i: (0, 0)),
            pl.BlockSpec((ndim, 1), lambda i: (0, 0)),
            pl.BlockSpec((1, 1), lambda i: (0, 0)),
        ],
        out_specs=pl.BlockSpec((_BM, gdim), lambda i: (i, 0)),
        out_shape=jax.ShapeDtypeStruct((n, gdim), jnp.float32),
    )(h, W_f, b_f.reshape(1, gdim), W_g, b_g.reshape(1, 1))


_C = 80


def _make_scatter(n, gdim):
    nchunk = n // _C
    cpw = (nchunk + 31) // 32
    rows_per_sub = NUM_GRAPHS // 16
    mesh = plsc.VectorSubcoreMesh(core_axis_name="c", subcore_axis_name="s")

    @functools.partial(
        pl.kernel,
        out_type=jax.ShapeDtypeStruct((2, NUM_GRAPHS, gdim), jnp.float32),
        mesh=mesh,
        scratch_types=[
            pltpu.VMEM((cpw, _C), jnp.int32),
            pltpu.VMEM((_C, gdim), jnp.float32),
            pltpu.VMEM((_C, gdim), jnp.float32),
            pltpu.VMEM_SHARED((NUM_GRAPHS, gdim), jnp.float32),
            pltpu.SemaphoreType.DMA,
            pltpu.SemaphoreType.DMA,
        ],
    )
    def _scatter(hg_hbm, idx2d_hbm, zeros_hbm, out_hbm,
                 idx_all, rows0, rows1, acc, sem0, sem1):
        cid = lax.axis_index("c")
        sid = lax.axis_index("s")
        wid = sid * 2 + cid

        pltpu.sync_copy(
            zeros_hbm.at[pl.ds(sid * rows_per_sub, rows_per_sub)],
            acc.at[pl.ds(sid * rows_per_sub, rows_per_sub)],
        )
        pltpu.sync_copy(
            idx2d_hbm.at[pl.ds(wid * cpw, cpw)],
            idx_all,
        )
        plsc.subcore_barrier()

        nt = jnp.clip(nchunk - wid * cpw, 0, cpw)

        def chunk_of(t):
            return wid * cpw + t

        def start_load(t, rows_v, sem):
            pltpu.async_copy(hg_hbm.at[pl.ds(chunk_of(t) * _C, _C)], rows_v, sem)

        def wait_load(t, rows_v, sem):
            pltpu.make_async_copy(
                hg_hbm.at[pl.ds(chunk_of(t) * _C, _C)], rows_v, sem
            ).wait()

        def process(t, rows_v, sem, rows_n, sem_n):
            wait_load(t, rows_v, sem)

            @pl.when(t + 1 < nt)
            def _():
                start_load(t + 1, rows_n, sem_n)

            pltpu.sync_copy(rows_v, acc.at[idx_all.at[t]], add=True)

        start_load(0, rows0, sem0)

        def body(t, carry):
            @pl.when(t % 2 == 0)
            def _():
                process(t, rows0, sem0, rows1, sem1)

            @pl.when(t % 2 == 1)
            def _():
                process(t, rows1, sem1, rows0, sem0)

            return carry

        lax.fori_loop(0, nt, body, 0)
        plsc.subcore_barrier()

        pltpu.sync_copy(
            acc.at[pl.ds(sid * rows_per_sub, rows_per_sub)],
            out_hbm.at[cid, pl.ds(sid * rows_per_sub, rows_per_sub)],
        )

    return _scatter


def _combine_body(p_ref, o_ref):
    o_ref[...] = p_ref[0] + p_ref[1]


def _combine(partials):
    _, ngraphs, gdim = partials.shape
    return pl.pallas_call(
        _combine_body,
        out_shape=jax.ShapeDtypeStruct((ngraphs, gdim), jnp.float32),
    )(partials)


def kernel(h, batch, W_f, b_f, W_g, b_g):
    n, _ = h.shape
    gdim = W_f.shape[1]
    hg = _dense_stage(h, W_f, b_f, W_g, b_g)
    nchunk = n // _C
    cpw = (nchunk + 31) // 32
    pad_rows = 32 * cpw - nchunk
    idx2d = batch.astype(jnp.int32).reshape(nchunk, _C)
    idx2d = jnp.concatenate(
        [idx2d, jnp.zeros((pad_rows, _C), jnp.int32)], axis=0
    )
    zeros = jnp.zeros((NUM_GRAPHS, gdim), jnp.float32)
    partials = _make_scatter(n, gdim)(hg, idx2d, zeros)
    return _combine(partials)


# speedup vs baseline: 3.6559x; 1.7104x over previous
import functools

import jax
import jax.numpy as jnp
from jax import lax
from jax.experimental import pallas as pl
from jax.experimental.pallas import tpu as pltpu
from jax.experimental.pallas import tpu_sc as plsc

NUM_GRAPHS = 1024

_C = 80
_GROWS = 32
_BM = _GROWS * _C


def _gate_body(h_ref, wgt_ref, bg_ref, g_ref):
    hb = h_ref[...]
    s = lax.dot_general(
        wgt_ref[...], hb, (((1,), (1,)), ((), ())),
        preferred_element_type=jnp.float32,
    )
    s = s + bg_ref[...]
    g_ref[0] = 1.0 / (1.0 + jnp.exp(-s))


def _gate_stage(h, W_g, b_g, nchunk_pad):
    n, ndim = h.shape
    nblocks = (nchunk_pad * _C) // _BM
    grid = (nblocks,)
    g3d = pl.pallas_call(
        _gate_body,
        grid=grid,
        in_specs=[
            pl.BlockSpec((_BM, ndim), lambda i: (i, 0)),
            pl.BlockSpec((1, ndim), lambda i: (0, 0)),
            pl.BlockSpec((1, 1), lambda i: (0, 0)),
        ],
        out_specs=pl.BlockSpec((1, 1, _BM), lambda i: (i, 0, 0)),
        out_shape=jax.ShapeDtypeStruct((nblocks, 1, _BM), jnp.float32),
    )(h, W_g.reshape(1, ndim), b_g.reshape(1, 1))
    return g3d.reshape(nchunk_pad, _C)


def _splat(vec, lane):
    lane_v = (jnp.full((16,), 0, jnp.int32) + lane)[:, None]
    return lax.gather(
        vec,
        lane_v,
        dimension_numbers=lax.GatherDimensionNumbers(
            offset_dims=(),
            collapsed_slice_dims=(0,),
            start_index_map=(0,),
        ),
        slice_sizes=(1,),
        mode=lax.GatherScatterMode.PROMISE_IN_BOUNDS,
    )


def _make_scatter(n, ndim):
    nchunk = n // _C
    cpw = (nchunk + 31) // 32
    rows_per_sub = NUM_GRAPHS // 16
    mesh = plsc.VectorSubcoreMesh(core_axis_name="c", subcore_axis_name="s")

    @functools.partial(
        pl.kernel,
        out_type=(
            jax.ShapeDtypeStruct((2, NUM_GRAPHS, ndim), jnp.float32),
            jax.ShapeDtypeStruct((2, NUM_GRAPHS), jnp.float32),
        ),
        mesh=mesh,
        scratch_types=[
            pltpu.VMEM((cpw, _C), jnp.int32),
            pltpu.VMEM((cpw, _C), jnp.float32),
            pltpu.VMEM((_C, ndim), jnp.float32),
            pltpu.VMEM((_C, ndim), jnp.float32),
            pltpu.VMEM((NUM_GRAPHS // 16,), jnp.float32),
            pltpu.VMEM_SHARED((NUM_GRAPHS, ndim), jnp.float32),
            pltpu.VMEM_SHARED((NUM_GRAPHS,), jnp.float32),
            pltpu.SemaphoreType.DMA,
            pltpu.SemaphoreType.DMA,
        ],
    )
    def _scatter(h_hbm, idx2d_hbm, g2d_hbm, zeros_hbm,
                 outh_hbm, outg_hbm,
                 idx_all, g_all, rows0, rows1, gbuf, acc, acc_g,
                 sem0, sem1):
        cid = lax.axis_index("c")
        sid = lax.axis_index("s")
        wid = sid * 2 + cid

        pltpu.sync_copy(
            zeros_hbm.at[pl.ds(sid * rows_per_sub, rows_per_sub)],
            acc.at[pl.ds(sid * rows_per_sub, rows_per_sub)],
        )
        for i in range(rows_per_sub // 16):
            gbuf[pl.ds(i * 16, 16)] = jnp.zeros((16,), jnp.float32)
        pltpu.sync_copy(gbuf, acc_g.at[pl.ds(sid * rows_per_sub, rows_per_sub)])
        pltpu.sync_copy(idx2d_hbm.at[pl.ds(wid * cpw, cpw)], idx_all)
        pltpu.sync_copy(g2d_hbm.at[pl.ds(wid * cpw, cpw)], g_all)
        plsc.subcore_barrier()

        nt = jnp.clip(nchunk - wid * cpw, 0, cpw)

        def start_load(t, rows_v, sem):
            row0 = (wid * cpw + t) * _C
            pltpu.async_copy(h_hbm.at[pl.ds(row0, _C)], rows_v, sem)

        def wait_load(t, rows_v, sem):
            row0 = (wid * cpw + t) * _C
            pltpu.make_async_copy(
                h_hbm.at[pl.ds(row0, _C)], rows_v, sem
            ).wait()

        def process(t, rows_v, sem, rows_n, sem_n):
            wait_load(t, rows_v, sem)

            @pl.when(t + 1 < nt)
            def _():
                start_load(t + 1, rows_n, sem_n)

            def scale_row(r, carry):
                grp = (r // 16) * 16
                gvec = g_all[t, pl.ds(grp, 16)]
                gs = _splat(gvec, r - grp)
                for j in range(ndim // 16):
                    rows_v[r, pl.ds(j * 16, 16)] = (
                        rows_v[r, pl.ds(j * 16, 16)] * gs
                    )
                return carry

            lax.fori_loop(0, _C, scale_row, 0)

            pltpu.sync_copy(rows_v, acc.at[idx_all.at[t]], add=True)
            pltpu.sync_copy(g_all.at[t], acc_g.at[idx_all.at[t]], add=True)

        start_load(0, rows0, sem0)

        def body(t, carry):
            @pl.when(t % 2 == 0)
            def _():
                process(t, rows0, sem0, rows1, sem1)

            @pl.when(t % 2 == 1)
            def _():
                process(t, rows1, sem1, rows0, sem0)

            return carry

        lax.fori_loop(0, nt, body, 0)
        plsc.subcore_barrier()

        pltpu.sync_copy(
            acc.at[pl.ds(sid * rows_per_sub, rows_per_sub)],
            outh_hbm.at[cid, pl.ds(sid * rows_per_sub, rows_per_sub)],
        )
        pltpu.sync_copy(
            acc_g.at[pl.ds(sid * rows_per_sub, rows_per_sub)], gbuf
        )
        pltpu.sync_copy(
            gbuf, outg_hbm.at[cid, pl.ds(sid * rows_per_sub, rows_per_sub)]
        )

    return _scatter


def _final_body(ph_ref, pg_ref, wf_ref, bf_ref, o_ref):
    pooled = ph_ref[0] + ph_ref[1]
    gsum = pg_ref[0:1, :] + pg_ref[1:2, :]
    bias = lax.dot_general(
        gsum, bf_ref[...], (((0,), (0,)), ((), ())),
        preferred_element_type=jnp.float32,
    )
    o_ref[...] = (
        jnp.dot(pooled, wf_ref[...], preferred_element_type=jnp.float32)
        + bias
    )


def _final(ph, pg, W_f, b_f):
    gdim = W_f.shape[1]
    return pl.pallas_call(
        _final_body,
        out_shape=jax.ShapeDtypeStruct((NUM_GRAPHS, gdim), jnp.float32),
    )(ph, pg, W_f, b_f.reshape(1, gdim))


def kernel(h, batch, W_f, b_f, W_g, b_g):
    n, ndim = h.shape
    nchunk = n // _C
    cpw = (nchunk + 31) // 32
    nchunk_pad = 32 * cpw
    idx2d = batch.astype(jnp.int32).reshape(nchunk, _C)
    idx2d = jnp.concatenate(
        [idx2d, jnp.zeros((nchunk_pad - nchunk, _C), jnp.int32)], axis=0
    )
    g2d = _gate_stage(h, W_g, b_g, nchunk_pad)
    zeros = jnp.zeros((NUM_GRAPHS, ndim), jnp.float32)
    ph, pg = _make_scatter(n, ndim)(h, idx2d, g2d, zeros)
    return _final(ph, pg, W_f, b_f)


# speedup vs baseline: 4.1194x; 1.1268x over previous
import functools

import jax
import jax.numpy as jnp
from jax import lax
from jax.experimental import pallas as pl
from jax.experimental.pallas import tpu as pltpu
from jax.experimental.pallas import tpu_sc as plsc

NUM_GRAPHS = 1024

_C = 80
_GROWS = 32
_BM = _GROWS * _C


def _gate_body(h_ref, wgt_ref, bg_ref, g_ref):
    hb = h_ref[...]
    s = lax.dot_general(
        wgt_ref[...], hb, (((1,), (1,)), ((), ())),
        preferred_element_type=jnp.float32,
    )
    s = s + bg_ref[...]
    g_ref[0] = 1.0 / (1.0 + jnp.exp(-s))


def _gate_stage(h, W_g, b_g, nchunk_pad):
    n, ndim = h.shape
    nblocks = (nchunk_pad * _C) // _BM
    grid = (nblocks,)
    g3d = pl.pallas_call(
        _gate_body,
        grid=grid,
        in_specs=[
            pl.BlockSpec((_BM, ndim), lambda i: (i, 0)),
            pl.BlockSpec((1, ndim), lambda i: (0, 0)),
            pl.BlockSpec((1, 1), lambda i: (0, 0)),
        ],
        out_specs=pl.BlockSpec((1, 1, _BM), lambda i: (i, 0, 0)),
        out_shape=jax.ShapeDtypeStruct((nblocks, 1, _BM), jnp.float32),
    )(h, W_g.reshape(1, ndim), b_g.reshape(1, 1))
    return g3d.reshape(nchunk_pad, _C)


def _splat(vec, lane):
    lane_v = (jnp.full((16,), 0, jnp.int32) + lane)[:, None]
    return lax.gather(
        vec,
        lane_v,
        dimension_numbers=lax.GatherDimensionNumbers(
            offset_dims=(),
            collapsed_slice_dims=(0,),
            start_index_map=(0,),
        ),
        slice_sizes=(1,),
        mode=lax.GatherScatterMode.PROMISE_IN_BOUNDS,
    )


def _make_scatter(n, ndim):
    nchunk = n // _C
    cpw = (nchunk + 31) // 32
    rows_per_sub = NUM_GRAPHS // 16
    mesh = plsc.VectorSubcoreMesh(core_axis_name="c", subcore_axis_name="s")

    @functools.partial(
        pl.kernel,
        out_type=(
            jax.ShapeDtypeStruct((2, NUM_GRAPHS, ndim), jnp.float32),
            jax.ShapeDtypeStruct((2, NUM_GRAPHS), jnp.float32),
        ),
        mesh=mesh,
        scratch_types=[
            pltpu.VMEM((cpw, _C), jnp.int32),
            pltpu.VMEM((cpw, _C), jnp.float32),
            pltpu.VMEM((_C, ndim), jnp.float32),
            pltpu.VMEM((_C, ndim), jnp.float32),
            pltpu.VMEM((NUM_GRAPHS // 16,), jnp.float32),
            pltpu.VMEM_SHARED((NUM_GRAPHS, ndim), jnp.float32),
            pltpu.VMEM_SHARED((NUM_GRAPHS,), jnp.float32),
            pltpu.SemaphoreType.DMA,
            pltpu.SemaphoreType.DMA,
        ],
    )
    def _scatter(h_hbm, idx2d_hbm, g2d_hbm, zeros_hbm,
                 outh_hbm, outg_hbm,
                 idx_all, g_all, rows0, rows1, gbuf, acc, acc_g,
                 sem0, sem1):
        cid = lax.axis_index("c")
        sid = lax.axis_index("s")
        wid = sid * 2 + cid

        pltpu.sync_copy(
            zeros_hbm.at[pl.ds(sid * rows_per_sub, rows_per_sub)],
            acc.at[pl.ds(sid * rows_per_sub, rows_per_sub)],
        )
        for i in range(rows_per_sub // 16):
            gbuf[pl.ds(i * 16, 16)] = jnp.zeros((16,), jnp.float32)
        pltpu.sync_copy(gbuf, acc_g.at[pl.ds(sid * rows_per_sub, rows_per_sub)])
        pltpu.sync_copy(idx2d_hbm.at[pl.ds(wid * cpw, cpw)], idx_all)
        pltpu.sync_copy(g2d_hbm.at[pl.ds(wid * cpw, cpw)], g_all)
        plsc.subcore_barrier()

        nt = jnp.clip(nchunk - wid * cpw, 0, cpw)

        def start_load(t, rows_v, sem):
            row0 = (wid * cpw + t) * _C
            pltpu.async_copy(h_hbm.at[pl.ds(row0, _C)], rows_v, sem)

        def wait_load(t, rows_v, sem):
            row0 = (wid * cpw + t) * _C
            pltpu.make_async_copy(
                h_hbm.at[pl.ds(row0, _C)], rows_v, sem
            ).wait()

        def process(t, rows_v, sem, rows_n, sem_n):
            wait_load(t, rows_v, sem)

            @pl.when(t + 1 < nt)
            def _():
                start_load(t + 1, rows_n, sem_n)

            @plsc.parallel_loop(0, _C, unroll=4)
            def scale_row(r):
                grp = (r // 16) * 16
                gvec = g_all[t, pl.ds(grp, 16)]
                gs = _splat(gvec, r - grp)
                for j in range(ndim // 16):
                    rows_v[r, pl.ds(j * 16, 16)] = (
                        rows_v[r, pl.ds(j * 16, 16)] * gs
                    )

            pltpu.sync_copy(rows_v, acc.at[idx_all.at[t]], add=True)
            pltpu.sync_copy(g_all.at[t], acc_g.at[idx_all.at[t]], add=True)

        start_load(0, rows0, sem0)

        def body(t, carry):
            @pl.when(t % 2 == 0)
            def _():
                process(t, rows0, sem0, rows1, sem1)

            @pl.when(t % 2 == 1)
            def _():
                process(t, rows1, sem1, rows0, sem0)

            return carry

        lax.fori_loop(0, nt, body, 0)
        plsc.subcore_barrier()

        pltpu.sync_copy(
            acc.at[pl.ds(sid * rows_per_sub, rows_per_sub)],
            outh_hbm.at[cid, pl.ds(sid * rows_per_sub, rows_per_sub)],
        )
        pltpu.sync_copy(
            acc_g.at[pl.ds(sid * rows_per_sub, rows_per_sub)], gbuf
        )
        pltpu.sync_copy(
            gbuf, outg_hbm.at[cid, pl.ds(sid * rows_per_sub, rows_per_sub)]
        )

    return _scatter


def _final_body(ph_ref, pg_ref, wf_ref, bf_ref, o_ref):
    pooled = ph_ref[0] + ph_ref[1]
    gsum = pg_ref[0:1, :] + pg_ref[1:2, :]
    bias = lax.dot_general(
        gsum, bf_ref[...], (((0,), (0,)), ((), ())),
        preferred_element_type=jnp.float32,
    )
    o_ref[...] = (
        jnp.dot(pooled, wf_ref[...], preferred_element_type=jnp.float32)
        + bias
    )


def _final(ph, pg, W_f, b_f):
    gdim = W_f.shape[1]
    return pl.pallas_call(
        _final_body,
        out_shape=jax.ShapeDtypeStruct((NUM_GRAPHS, gdim), jnp.float32),
    )(ph, pg, W_f, b_f.reshape(1, gdim))


def kernel(h, batch, W_f, b_f, W_g, b_g):
    n, ndim = h.shape
    nchunk = n // _C
    cpw = (nchunk + 31) // 32
    nchunk_pad = 32 * cpw
    idx2d = batch.astype(jnp.int32).reshape(nchunk, _C)
    idx2d = jnp.concatenate(
        [idx2d, jnp.zeros((nchunk_pad - nchunk, _C), jnp.int32)], axis=0
    )
    g2d = _gate_stage(h, W_g, b_g, nchunk_pad)
    zeros = jnp.zeros((NUM_GRAPHS, ndim), jnp.float32)
    ph, pg = _make_scatter(n, ndim)(h, idx2d, g2d, zeros)
    return _final(ph, pg, W_f, b_f)


# speedup vs baseline: 4.4863x; 1.0891x over previous
import functools

import jax
import jax.numpy as jnp
from jax import lax
from jax.experimental import pallas as pl
from jax.experimental.pallas import tpu as pltpu
from jax.experimental.pallas import tpu_sc as plsc

NUM_GRAPHS = 1024

_C = 80
_GROWS = 32
_BM = _GROWS * _C


def _gate_body(h_ref, wgt_ref, bg_ref, g_ref):
    hb = h_ref[...]
    s = lax.dot_general(
        wgt_ref[...], hb, (((1,), (1,)), ((), ())),
        preferred_element_type=jnp.float32,
    )
    s = s + bg_ref[...]
    g_ref[0] = 1.0 / (1.0 + jnp.exp(-s))


def _gate_stage(h, W_g, b_g, block_base, nblocks):
    _, ndim = h.shape
    g3d = pl.pallas_call(
        _gate_body,
        grid=(nblocks,),
        in_specs=[
            pl.BlockSpec((_BM, ndim), lambda i: (i + block_base, 0)),
            pl.BlockSpec((1, ndim), lambda i: (0, 0)),
            pl.BlockSpec((1, 1), lambda i: (0, 0)),
        ],
        out_specs=pl.BlockSpec((1, 1, _BM), lambda i: (i, 0, 0)),
        out_shape=jax.ShapeDtypeStruct((nblocks, 1, _BM), jnp.float32),
    )(h, W_g.reshape(1, ndim), b_g.reshape(1, 1))
    return g3d.reshape(nblocks * _BM)


def _splat(vec, lane):
    lane_v = (jnp.full((16,), 0, jnp.int32) + lane)[:, None]
    return lax.gather(
        vec,
        lane_v,
        dimension_numbers=lax.GatherDimensionNumbers(
            offset_dims=(),
            collapsed_slice_dims=(0,),
            start_index_map=(0,),
        ),
        slice_sizes=(1,),
        mode=lax.GatherScatterMode.PROMISE_IN_BOUNDS,
    )


def _make_scatter(ndim, chunk_base, nchunk_part, g_row_base):
    rows_per_sub = NUM_GRAPHS // 16
    mesh = plsc.VectorSubcoreMesh(core_axis_name="c", subcore_axis_name="s")

    @functools.partial(
        pl.kernel,
        out_type=(
            jax.ShapeDtypeStruct((2, NUM_GRAPHS, ndim), jnp.float32),
            jax.ShapeDtypeStruct((2, NUM_GRAPHS), jnp.float32),
        ),
        mesh=mesh,
        scratch_types=[
            pltpu.VMEM((_C,), jnp.int32),
            pltpu.VMEM((_C,), jnp.int32),
            pltpu.VMEM((_C,), jnp.float32),
            pltpu.VMEM((_C,), jnp.float32),
            pltpu.VMEM((_C, ndim), jnp.float32),
            pltpu.VMEM((_C, ndim), jnp.float32),
            pltpu.VMEM((NUM_GRAPHS // 16,), jnp.float32),
            pltpu.VMEM_SHARED((NUM_GRAPHS, ndim), jnp.float32),
            pltpu.VMEM_SHARED((NUM_GRAPHS,), jnp.float32),
            pltpu.SemaphoreType.DMA,
            pltpu.SemaphoreType.DMA,
        ],
    )
    def _scatter(h_hbm, batch_hbm, g1d_hbm, zeros_hbm,
                 outh_hbm, outg_hbm,
                 idx0, idx1, g0, g1, rows0, rows1, gbuf, acc, acc_g,
                 sem0, sem1):
        cid = lax.axis_index("c")
        sid = lax.axis_index("s")
        wid = sid * 2 + cid

        pltpu.sync_copy(
            zeros_hbm.at[pl.ds(sid * rows_per_sub, rows_per_sub)],
            acc.at[pl.ds(sid * rows_per_sub, rows_per_sub)],
        )
        for i in range(rows_per_sub // 16):
            gbuf[pl.ds(i * 16, 16)] = jnp.zeros((16,), jnp.float32)
        pltpu.sync_copy(gbuf, acc_g.at[pl.ds(sid * rows_per_sub, rows_per_sub)])
        plsc.subcore_barrier()

        nt = (nchunk_part - wid + 31) // 32

        def srcs(t, idx_v, g_v, rows_v):
            c = chunk_base + wid + t * 32
            row0 = c * _C
            return (
                (batch_hbm.at[pl.ds(row0, _C)], idx_v),
                (g1d_hbm.at[pl.ds(row0 - g_row_base, _C)], g_v),
                (h_hbm.at[pl.ds(row0, _C)], rows_v),
            )

        def start_load(t, idx_v, g_v, rows_v, sem):
            for src, dst in srcs(t, idx_v, g_v, rows_v):
                pltpu.async_copy(src, dst, sem)

        def wait_load(t, idx_v, g_v, rows_v, sem):
            for src, dst in srcs(t, idx_v, g_v, rows_v):
                pltpu.make_async_copy(src, dst, sem).wait()

        def process(t, idx_v, g_v, rows_v, sem, idx_n, g_n, rows_n, sem_n):
            wait_load(t, idx_v, g_v, rows_v, sem)

            @pl.when(t + 1 < nt)
            def _():
                start_load(t + 1, idx_n, g_n, rows_n, sem_n)

            @plsc.parallel_loop(0, _C, unroll=4)
            def scale_row(r):
                grp = (r // 16) * 16
                gvec = g_v[pl.ds(grp, 16)]
                gs = _splat(gvec, r - grp)
                for j in range(ndim // 16):
                    rows_v[r, pl.ds(j * 16, 16)] = (
                        rows_v[r, pl.ds(j * 16, 16)] * gs
                    )

            pltpu.sync_copy(rows_v, acc.at[idx_v], add=True)
            pltpu.sync_copy(g_v, acc_g.at[idx_v], add=True)

        start_load(0, idx0, g0, rows0, sem0)

        def body(t, carry):
            @pl.when(t % 2 == 0)
            def _():
                process(t, idx0, g0, rows0, sem0, idx1, g1, rows1, sem1)

            @pl.when(t % 2 == 1)
            def _():
                process(t, idx1, g1, rows1, sem1, idx0, g0, rows0, sem0)

            return carry

        lax.fori_loop(0, nt, body, 0)
        plsc.subcore_barrier()

        pltpu.sync_copy(
            acc.at[pl.ds(sid * rows_per_sub, rows_per_sub)],
            outh_hbm.at[cid, pl.ds(sid * rows_per_sub, rows_per_sub)],
        )
        pltpu.sync_copy(
            acc_g.at[pl.ds(sid * rows_per_sub, rows_per_sub)], gbuf
        )
        pltpu.sync_copy(
            gbuf, outg_hbm.at[cid, pl.ds(sid * rows_per_sub, rows_per_sub)]
        )

    return _scatter


def _final_body(pha_ref, phb_ref, pga_ref, pgb_ref, wf_ref, bf_ref, o_ref):
    pooled = pha_ref[0] + pha_ref[1] + phb_ref[0] + phb_ref[1]
    gsum = (
        pga_ref[0:1, :] + pga_ref[1:2, :] + pgb_ref[0:1, :] + pgb_ref[1:2, :]
    )
    bias = lax.dot_general(
        gsum, bf_ref[...], (((0,), (0,)), ((), ())),
        preferred_element_type=jnp.float32,
    )
    o_ref[...] = (
        jnp.dot(pooled, wf_ref[...], preferred_element_type=jnp.float32)
        + bias
    )


def _final(pha, phb, pga, pgb, W_f, b_f):
    gdim = W_f.shape[1]
    return pl.pallas_call(
        _final_body,
        out_shape=jax.ShapeDtypeStruct((NUM_GRAPHS, gdim), jnp.float32),
    )(pha, phb, pga, pgb, W_f, b_f.reshape(1, gdim))


def kernel(h, batch, W_f, b_f, W_g, b_g):
    n, ndim = h.shape
    nchunk = n // _C
    half_chunks = nchunk // 2
    half_rows = half_chunks * _C
    blocks0 = pl.cdiv(half_rows, _BM)
    b1_base = half_rows // _BM
    blocks1 = pl.cdiv(n, _BM) - b1_base

    idx = batch.astype(jnp.int32)
    zeros = jnp.zeros((NUM_GRAPHS, ndim), jnp.float32)

    g0 = _gate_stage(h, W_g, b_g, 0, blocks0)
    g1 = _gate_stage(h, W_g, b_g, b1_base, blocks1)
    pha, pga = _make_scatter(ndim, 0, half_chunks, 0)(h, idx, g0, zeros)
    phb, pgb = _make_scatter(ndim, half_chunks, nchunk - half_chunks,
                             b1_base * _BM)(h, idx, g1, zeros)
    return _final(pha, phb, pga, pgb, W_f, b_f)
